# Initial kernel scaffold; baseline (speedup 1.0000x reference)
import jax
import jax.numpy as jnp
from jax.experimental import pallas as pl


def kernel(x, edge_index, W0, b0, W1, b1, M0, mb0, M1, mb1, M2, mb2):
    raise NotImplementedError("write your pallas kernel here")


# speedup vs baseline: 13.4885x; 13.4885x over previous
import functools

import jax
import jax.numpy as jnp
from jax import lax
from jax.experimental import pallas as pl
from jax.experimental.pallas import tpu as pltpu
from jax.experimental.pallas import tpu_sc as plsc

_N = 10000
_E = 320000
_D = 128
_H = 128

_NC = 2
_NS = 16
_NW = _NC * _NS
_EPT = _E // _NW
_CHUNK = 80
_NCHUNK = _EPT // _CHUNK
_RPT = 632
_RPT_LAST = _N - 15 * _RPT

_mesh = plsc.VectorSubcoreMesh(core_axis_name="c", subcore_axis_name="s")


@functools.partial(
    pl.kernel,
    out_type=jax.ShapeDtypeStruct((_NC * _N,), jnp.float32),
    mesh=_mesh,
    scratch_types=[
        pltpu.VMEM((_CHUNK,), jnp.int32),
        pltpu.VMEM((_CHUNK,), jnp.float32),
        pltpu.VMEM((2000,), jnp.float32),
        pltpu.VMEM_SHARED((_N,), jnp.float32),
    ],
)
def _deg_kernel(dst_hbm, out_hbm, idx_v, ones_v, zbuf_v, acc_sh):
    c = lax.axis_index("c")
    s = lax.axis_index("s")
    wid = c * _NS + s
    @pl.when(s < 5)
    def _():
        def zfill(i, carry):
            zbuf_v[pl.ds(i * 16, 16)] = jnp.zeros((16,), jnp.float32)
            return carry
        lax.fori_loop(0, 2000 // 16, zfill, 0)
        pltpu.sync_copy(zbuf_v, acc_sh.at[pl.ds(s * 2000, 2000)])
    for k in range(_CHUNK // 16):
        ones_v[pl.ds(k * 16, 16)] = jnp.ones((16,), jnp.float32)
    plsc.subcore_barrier()

    def body(j, carry):
        base = pl.multiple_of(wid * _EPT + j * _CHUNK, 8)
        pltpu.sync_copy(dst_hbm.at[pl.ds(base, _CHUNK)], idx_v)
        pltpu.sync_copy(ones_v, acc_sh.at[idx_v], add=True)
        return carry

    lax.fori_loop(0, _NCHUNK, body, 0)
    plsc.subcore_barrier()
    @pl.when(s < 5)
    def _():
        pltpu.sync_copy(acc_sh.at[pl.ds(s * 2000, 2000)], zbuf_v)
        pltpu.sync_copy(zbuf_v, out_hbm.at[pl.ds(c * _N + s * 2000, 2000)])


@functools.partial(
    pl.kernel,
    out_type=jax.ShapeDtypeStruct((_NC, _N, _H), jnp.float32),
    mesh=_mesh,
    scratch_types=[
        pltpu.VMEM((_CHUNK,), jnp.int32),
        pltpu.VMEM((_CHUNK,), jnp.int32),
        pltpu.VMEM((_CHUNK, _H), jnp.float32),
        pltpu.VMEM_SHARED((_N, _H), jnp.float32),
        pltpu.SemaphoreType.DMA,
    ],
)
def _agg_kernel(y_hbm, src_hbm, dst_hbm, out_hbm,
                src_v, dst_v, rows_v, acc_sh, sem):
    c = lax.axis_index("c")
    s = lax.axis_index("s")
    wid = c * _NS + s
    r0 = s * _RPT
    def zrow(i, carry):
        for k in range(_H // 16):
            rows_v[i, pl.ds(k * 16, 16)] = jnp.zeros((16,), jnp.float32)
        return carry
    lax.fori_loop(0, _CHUNK, zrow, 0)
    @pl.when(s < 15)
    def _():
        for t in range(_RPT // _CHUNK):
            pltpu.sync_copy(rows_v, acc_sh.at[pl.ds(r0 + t * _CHUNK, _CHUNK)])
        pltpu.sync_copy(rows_v.at[pl.ds(0, _RPT % _CHUNK)],
                        acc_sh.at[pl.ds(r0 + _RPT - _RPT % _CHUNK,
                                        _RPT % _CHUNK)])
    @pl.when(s == 15)
    def _():
        for t in range(_RPT_LAST // _CHUNK):
            pltpu.sync_copy(rows_v, acc_sh.at[pl.ds(r0 + t * _CHUNK, _CHUNK)])
        pltpu.sync_copy(rows_v.at[pl.ds(0, _RPT_LAST % _CHUNK)],
                        acc_sh.at[pl.ds(r0 + _RPT_LAST - _RPT_LAST % _CHUNK,
                                        _RPT_LAST % _CHUNK)])
    plsc.subcore_barrier()

    def body(j, carry):
        base = pl.multiple_of(wid * _EPT + j * _CHUNK, 8)
        pltpu.sync_copy(src_hbm.at[pl.ds(base, _CHUNK)], src_v)
        pltpu.sync_copy(dst_hbm.at[pl.ds(base, _CHUNK)], dst_v)
        pltpu.async_copy(y_hbm.at[src_v], rows_v, sem).wait()
        pltpu.sync_copy(rows_v, acc_sh.at[dst_v], add=True)
        return carry

    lax.fori_loop(0, _NCHUNK, body, 0)
    plsc.subcore_barrier()

    def copy_out(nrows):
        for t in range(nrows // _CHUNK):
            pltpu.sync_copy(acc_sh.at[pl.ds(r0 + t * _CHUNK, _CHUNK)], rows_v)
            pltpu.sync_copy(rows_v, out_hbm.at[c, pl.ds(r0 + t * _CHUNK,
                                                        _CHUNK)])
        rem = nrows % _CHUNK
        pltpu.sync_copy(acc_sh.at[pl.ds(r0 + nrows - rem, rem)],
                        rows_v.at[pl.ds(0, rem)])
        pltpu.sync_copy(rows_v.at[pl.ds(0, rem)],
                        out_hbm.at[c, pl.ds(r0 + nrows - rem, rem)])

    @pl.when(s < 15)
    def _():
        copy_out(_RPT)
    @pl.when(s == 15)
    def _():
        copy_out(_RPT_LAST)


_R = 1000
_G = _N // _R


def _lin0_body(x_ref, w_ref, degp_ref, y_ref, dis_ref):
    deg = degp_ref[:, 0:1] + degp_ref[:, 1:2] + 1.0
    dis = lax.rsqrt(deg)
    dis_ref[...] = dis
    y_ref[...] = jnp.dot(x_ref[...], w_ref[...],
                         preferred_element_type=jnp.float32) * dis


def _lin0(x, w, degp_t):
    return pl.pallas_call(
        _lin0_body,
        grid=(_G,),
        in_specs=[
            pl.BlockSpec((_R, _D), lambda i: (i, 0)),
            pl.BlockSpec((_D, _H), lambda i: (0, 0)),
            pl.BlockSpec((_R, 2), lambda i: (i, 0)),
        ],
        out_specs=[
            pl.BlockSpec((_R, _H), lambda i: (i, 0)),
            pl.BlockSpec((_R, 1), lambda i: (i, 0)),
        ],
        out_shape=[
            jax.ShapeDtypeStruct((_N, _H), jnp.float32),
            jax.ShapeDtypeStruct((_N, 1), jnp.float32),
        ],
    )(x, w, degp_t)


def _mid_body(p_ref, y_ref, dis_ref, b_ref, w_ref, y1_ref):
    agg = p_ref[0] + p_ref[1] + y_ref[...]
    dis = dis_ref[...]
    h = jnp.maximum(agg * dis + b_ref[...], 0.0)
    y1_ref[...] = jnp.dot(h, w_ref[...],
                          preferred_element_type=jnp.float32) * dis


def _mid(p, y, dis, b, w):
    return pl.pallas_call(
        _mid_body,
        grid=(_G,),
        in_specs=[
            pl.BlockSpec((2, _R, _H), lambda i: (0, i, 0)),
            pl.BlockSpec((_R, _H), lambda i: (i, 0)),
            pl.BlockSpec((_R, 1), lambda i: (i, 0)),
            pl.BlockSpec((1, _H), lambda i: (0, 0)),
            pl.BlockSpec((_H, _H), lambda i: (0, 0)),
        ],
        out_specs=pl.BlockSpec((_R, _H), lambda i: (i, 0)),
        out_shape=jax.ShapeDtypeStruct((_N, _H), jnp.float32),
    )(p, y, dis, b, w)


def _head_body(q_ref, y_ref, dis_ref, b1_ref, m0_ref, mb0_ref, m1_ref,
               mb1_ref, m2_ref, mb2_ref, o_ref):
    agg = q_ref[0] + q_ref[1] + y_ref[...]
    h = jnp.maximum(agg * dis_ref[...] + b1_ref[...], 0.0)
    h = jnp.maximum(jnp.dot(h, m0_ref[...],
                            preferred_element_type=jnp.float32) + mb0_ref[...], 0.0)
    h = jnp.maximum(jnp.dot(h, m1_ref[...],
                            preferred_element_type=jnp.float32) + mb1_ref[...], 0.0)
    o = jnp.dot(h, m2_ref[...], preferred_element_type=jnp.float32) + mb2_ref[...]
    o_ref[...] = jax.nn.sigmoid(o)


def _head(q, y, dis, b1, m0, mb0, m1, mb1, m2, mb2):
    return pl.pallas_call(
        _head_body,
        grid=(_G,),
        in_specs=[
            pl.BlockSpec((2, _R, _H), lambda i: (0, i, 0)),
            pl.BlockSpec((_R, _H), lambda i: (i, 0)),
            pl.BlockSpec((_R, 1), lambda i: (i, 0)),
            pl.BlockSpec((1, _H), lambda i: (0, 0)),
            pl.BlockSpec((_H, _H), lambda i: (0, 0)),
            pl.BlockSpec((1, _H), lambda i: (0, 0)),
            pl.BlockSpec((_H, _H), lambda i: (0, 0)),
            pl.BlockSpec((1, _H), lambda i: (0, 0)),
            pl.BlockSpec((_H, 1), lambda i: (0, 0)),
            pl.BlockSpec((1, 1), lambda i: (0, 0)),
        ],
        out_specs=pl.BlockSpec((_R, 1), lambda i: (i, 0)),
        out_shape=jax.ShapeDtypeStruct((_N, 1), jnp.float32),
    )(q, y, dis, b1, m0, mb0, m1, mb1, m2, mb2)


def kernel(x, edge_index, W0, b0, W1, b1, M0, mb0, M1, mb1, M2, mb2):
    ei = edge_index.astype(jnp.int32)
    src = ei[0]
    dst = ei[1]
    degp = _deg_kernel(dst)
    degp_t = degp.reshape(_NC, _N).T

    y0, dis = _lin0(x, W0, degp_t)
    p = _agg_kernel(y0, src, dst)
    y1 = _mid(p, y0, dis, b0.reshape(1, _H), W1)
    q = _agg_kernel(y1, src, dst)
    out = _head(q, y1, dis, b1.reshape(1, _H), M0, mb0.reshape(1, _H),
                M1, mb1.reshape(1, _H), M2, mb2.reshape(1, 1))
    return out


# speedup vs baseline: 24.8694x; 1.8437x over previous
import functools

import jax
import jax.numpy as jnp
from jax import lax
from jax.experimental import pallas as pl
from jax.experimental.pallas import tpu as pltpu
from jax.experimental.pallas import tpu_sc as plsc

_N = 10000
_E = 320000
_D = 128
_H = 128

_NC = 2
_NS = 16
_NW = _NC * _NS
_EPT = _E // _NW
_CHUNK = 80
_NCHUNK = _EPT // _CHUNK
_RPT = 632
_RPT_LAST = _N - 15 * _RPT

_mesh = plsc.VectorSubcoreMesh(core_axis_name="c", subcore_axis_name="s")

@functools.partial(
    pl.kernel,
    out_type=jax.ShapeDtypeStruct((_NC * _N,), jnp.float32),
    mesh=_mesh,
    scratch_types=[
        pltpu.VMEM((_CHUNK,), jnp.int32),
        pltpu.VMEM((_CHUNK,), jnp.float32),
        pltpu.VMEM((2000,), jnp.float32),
        pltpu.VMEM_SHARED((_N,), jnp.float32),
    ],
)
def _deg_kernel(dst_hbm, out_hbm, idx_v, ones_v, zbuf_v, acc_sh):
    c = lax.axis_index("c")
    s = lax.axis_index("s")
    wid = c * _NS + s
    @pl.when(s < 5)
    def _():
        def zfill(i, carry):
            zbuf_v[pl.ds(i * 16, 16)] = jnp.zeros((16,), jnp.float32)
            return carry
        lax.fori_loop(0, 2000 // 16, zfill, 0)
        pltpu.sync_copy(zbuf_v, acc_sh.at[pl.ds(s * 2000, 2000)])
    for k in range(_CHUNK // 16):
        ones_v[pl.ds(k * 16, 16)] = jnp.ones((16,), jnp.float32)
    plsc.subcore_barrier()

    def body(j, carry):
        base = pl.multiple_of(wid * _EPT + j * _CHUNK, 8)
        pltpu.sync_copy(dst_hbm.at[pl.ds(base, _CHUNK)], idx_v)
        pltpu.sync_copy(ones_v, acc_sh.at[idx_v], add=True)
        return carry

    lax.fori_loop(0, _NCHUNK, body, 0)
    plsc.subcore_barrier()
    @pl.when(s < 5)
    def _():
        pltpu.sync_copy(acc_sh.at[pl.ds(s * 2000, 2000)], zbuf_v)
        pltpu.sync_copy(zbuf_v, out_hbm.at[pl.ds(c * _N + s * 2000, 2000)])


_GK = 4
_NGRP = _NCHUNK // _GK


@functools.partial(
    pl.kernel,
    out_type=jax.ShapeDtypeStruct((_NC, _N, _H), jnp.float32),
    mesh=_mesh,
    scratch_types=[
        pltpu.VMEM((_GK, _CHUNK), jnp.int32),
        pltpu.VMEM((_GK, _CHUNK), jnp.int32),
        pltpu.VMEM((_GK * _CHUNK, _H), jnp.float32),
        pltpu.VMEM_SHARED((_N, _H), jnp.float32),
        pltpu.SemaphoreType.DMA,
        pltpu.SemaphoreType.DMA,
    ],
)
def _agg_kernel(y_hbm, src_hbm, dst_hbm, out_hbm,
                src_v, dst_v, rows_v, acc_sh, isem, gsem):
    c = lax.axis_index("c")
    s = lax.axis_index("s")
    wid = c * _NS + s
    r0 = s * _RPT
    def zrow(i, carry):
        for k in range(_H // 16):
            rows_v[i, pl.ds(k * 16, 16)] = jnp.zeros((16,), jnp.float32)
        return carry
    lax.fori_loop(0, _CHUNK, zrow, 0)
    zslot = rows_v.at[pl.ds(0, _CHUNK)]

    @pl.when(s < 15)
    def _():
        for t in range(_RPT // _CHUNK):
            pltpu.sync_copy(zslot, acc_sh.at[pl.ds(r0 + t * _CHUNK, _CHUNK)])
        pltpu.sync_copy(rows_v.at[pl.ds(0, _RPT % _CHUNK)],
                        acc_sh.at[pl.ds(r0 + _RPT - _RPT % _CHUNK,
                                        _RPT % _CHUNK)])
    @pl.when(s == 15)
    def _():
        for t in range(_RPT_LAST // _CHUNK):
            pltpu.sync_copy(zslot, acc_sh.at[pl.ds(r0 + t * _CHUNK, _CHUNK)])
        pltpu.sync_copy(rows_v.at[pl.ds(0, _RPT_LAST % _CHUNK)],
                        acc_sh.at[pl.ds(r0 + _RPT_LAST - _RPT_LAST % _CHUNK,
                                        _RPT_LAST % _CHUNK)])
    plsc.subcore_barrier()

    def slot(k):
        return rows_v.at[pl.ds(k * _CHUNK, _CHUNK)]

    def ebase(j):
        return pl.multiple_of(wid * _EPT + j * _CHUNK, 8)

    def body(g, carry):
        jb = g * _GK
        idescs = []
        for k in range(_GK):
            idescs.append(pltpu.async_copy(
                src_hbm.at[pl.ds(ebase(jb + k), _CHUNK)], src_v.at[k], isem))
            idescs.append(pltpu.async_copy(
                dst_hbm.at[pl.ds(ebase(jb + k), _CHUNK)], dst_v.at[k], isem))
        gdescs = []
        for k in range(_GK):
            idescs[2 * k].wait()
            gdescs.append(
                pltpu.async_copy(y_hbm.at[src_v.at[k]], slot(k), gsem))
        for k in range(_GK):
            gdescs[k].wait()
            idescs[2 * k + 1].wait()
            pltpu.sync_copy(slot(k), acc_sh.at[dst_v.at[k]], add=True)
        return carry

    lax.fori_loop(0, _NGRP, body, 0)
    for j in range(_NGRP * _GK, _NCHUNK):
        pltpu.sync_copy(src_hbm.at[pl.ds(ebase(j), _CHUNK)], src_v.at[0])
        pltpu.sync_copy(dst_hbm.at[pl.ds(ebase(j), _CHUNK)], dst_v.at[0])
        pltpu.async_copy(y_hbm.at[src_v.at[0]], slot(0), gsem).wait()
        pltpu.sync_copy(slot(0), acc_sh.at[dst_v.at[0]], add=True)
    plsc.subcore_barrier()

    def copy_out(nrows):
        stage = rows_v.at[pl.ds(0, _CHUNK)]
        for t in range(nrows // _CHUNK):
            pltpu.sync_copy(acc_sh.at[pl.ds(r0 + t * _CHUNK, _CHUNK)], stage)
            pltpu.sync_copy(stage, out_hbm.at[c, pl.ds(r0 + t * _CHUNK,
                                                       _CHUNK)])
        rem = nrows % _CHUNK
        pltpu.sync_copy(acc_sh.at[pl.ds(r0 + nrows - rem, rem)],
                        rows_v.at[pl.ds(0, rem)])
        pltpu.sync_copy(rows_v.at[pl.ds(0, rem)],
                        out_hbm.at[c, pl.ds(r0 + nrows - rem, rem)])

    @pl.when(s < 15)
    def _():
        copy_out(_RPT)
    @pl.when(s == 15)
    def _():
        copy_out(_RPT_LAST)


_R = 1000
_G = _N // _R


def _lin0_body(x_ref, w_ref, degp_ref, y_ref, dis_ref):
    deg = degp_ref[:, 0:1] + degp_ref[:, 1:2] + 1.0
    dis = lax.rsqrt(deg)
    dis_ref[...] = dis
    y_ref[...] = jnp.dot(x_ref[...], w_ref[...],
                         preferred_element_type=jnp.float32) * dis


def _lin0(x, w, degp_t):
    return pl.pallas_call(
        _lin0_body,
        grid=(_G,),
        in_specs=[
            pl.BlockSpec((_R, _D), lambda i: (i, 0)),
            pl.BlockSpec((_D, _H), lambda i: (0, 0)),
            pl.BlockSpec((_R, 2), lambda i: (i, 0)),
        ],
        out_specs=[
            pl.BlockSpec((_R, _H), lambda i: (i, 0)),
            pl.BlockSpec((_R, 1), lambda i: (i, 0)),
        ],
        out_shape=[
            jax.ShapeDtypeStruct((_N, _H), jnp.float32),
            jax.ShapeDtypeStruct((_N, 1), jnp.float32),
        ],
    )(x, w, degp_t)


def _mid_body(p_ref, y_ref, dis_ref, b_ref, w_ref, y1_ref):
    agg = p_ref[0] + p_ref[1] + y_ref[...]
    dis = dis_ref[...]
    h = jnp.maximum(agg * dis + b_ref[...], 0.0)
    y1_ref[...] = jnp.dot(h, w_ref[...],
                          preferred_element_type=jnp.float32) * dis


def _mid(p, y, dis, b, w):
    return pl.pallas_call(
        _mid_body,
        grid=(_G,),
        in_specs=[
            pl.BlockSpec((2, _R, _H), lambda i: (0, i, 0)),
            pl.BlockSpec((_R, _H), lambda i: (i, 0)),
            pl.BlockSpec((_R, 1), lambda i: (i, 0)),
            pl.BlockSpec((1, _H), lambda i: (0, 0)),
            pl.BlockSpec((_H, _H), lambda i: (0, 0)),
        ],
        out_specs=pl.BlockSpec((_R, _H), lambda i: (i, 0)),
        out_shape=jax.ShapeDtypeStruct((_N, _H), jnp.float32),
    )(p, y, dis, b, w)


def _head_body(q_ref, y_ref, dis_ref, b1_ref, m0_ref, mb0_ref, m1_ref,
               mb1_ref, m2_ref, mb2_ref, o_ref):
    agg = q_ref[0] + q_ref[1] + y_ref[...]
    h = jnp.maximum(agg * dis_ref[...] + b1_ref[...], 0.0)
    h = jnp.maximum(jnp.dot(h, m0_ref[...],
                            preferred_element_type=jnp.float32) + mb0_ref[...], 0.0)
    h = jnp.maximum(jnp.dot(h, m1_ref[...],
                            preferred_element_type=jnp.float32) + mb1_ref[...], 0.0)
    o = jnp.dot(h, m2_ref[...], preferred_element_type=jnp.float32) + mb2_ref[...]
    o_ref[...] = jax.nn.sigmoid(o)


def _head(q, y, dis, b1, m0, mb0, m1, mb1, m2, mb2):
    return pl.pallas_call(
        _head_body,
        grid=(_G,),
        in_specs=[
            pl.BlockSpec((2, _R, _H), lambda i: (0, i, 0)),
            pl.BlockSpec((_R, _H), lambda i: (i, 0)),
            pl.BlockSpec((_R, 1), lambda i: (i, 0)),
            pl.BlockSpec((1, _H), lambda i: (0, 0)),
            pl.BlockSpec((_H, _H), lambda i: (0, 0)),
            pl.BlockSpec((1, _H), lambda i: (0, 0)),
            pl.BlockSpec((_H, _H), lambda i: (0, 0)),
            pl.BlockSpec((1, _H), lambda i: (0, 0)),
            pl.BlockSpec((_H, 1), lambda i: (0, 0)),
            pl.BlockSpec((1, 1), lambda i: (0, 0)),
        ],
        out_specs=pl.BlockSpec((_R, 1), lambda i: (i, 0)),
        out_shape=jax.ShapeDtypeStruct((_N, 1), jnp.float32),
    )(q, y, dis, b1, m0, mb0, m1, mb1, m2, mb2)


def kernel(x, edge_index, W0, b0, W1, b1, M0, mb0, M1, mb1, M2, mb2):
    ei = edge_index.astype(jnp.int32)
    src = ei[0]
    dst = ei[1]
    degp = _deg_kernel(dst)
    degp_t = degp.reshape(_NC, _N).T

    y0, dis = _lin0(x, W0, degp_t)
    p = _agg_kernel(y0, src, dst)
    y1 = _mid(p, y0, dis, b0.reshape(1, _H), W1)
    q = _agg_kernel(y1, src, dst)
    out = _head(q, y1, dis, b1.reshape(1, _H), M0, mb0.reshape(1, _H),
                M1, mb1.reshape(1, _H), M2, mb2.reshape(1, 1))
    return out


# speedup vs baseline: 25.1361x; 1.0107x over previous
import functools

import jax
import jax.numpy as jnp
from jax import lax
from jax.experimental import pallas as pl
from jax.experimental.pallas import tpu as pltpu
from jax.experimental.pallas import tpu_sc as plsc

_N = 10000
_E = 320000
_D = 128
_H = 128

_NC = 2
_NS = 16
_NW = _NC * _NS
_EPT = _E // _NW
_CHUNK = 80
_NCHUNK = _EPT // _CHUNK
_RPT = 632
_RPT_LAST = _N - 15 * _RPT

_mesh = plsc.VectorSubcoreMesh(core_axis_name="c", subcore_axis_name="s")

@functools.partial(
    pl.kernel,
    out_type=jax.ShapeDtypeStruct((_NC * _N,), jnp.float32),
    mesh=_mesh,
    scratch_types=[
        pltpu.VMEM((_CHUNK,), jnp.int32),
        pltpu.VMEM((_CHUNK,), jnp.float32),
        pltpu.VMEM((2000,), jnp.float32),
        pltpu.VMEM_SHARED((_N,), jnp.float32),
    ],
)
def _deg_kernel(dst_hbm, out_hbm, idx_v, ones_v, zbuf_v, acc_sh):
    c = lax.axis_index("c")
    s = lax.axis_index("s")
    wid = c * _NS + s
    @pl.when(s < 5)
    def _():
        def zfill(i, carry):
            zbuf_v[pl.ds(i * 16, 16)] = jnp.zeros((16,), jnp.float32)
            return carry
        lax.fori_loop(0, 2000 // 16, zfill, 0)
        pltpu.sync_copy(zbuf_v, acc_sh.at[pl.ds(s * 2000, 2000)])
    for k in range(_CHUNK // 16):
        ones_v[pl.ds(k * 16, 16)] = jnp.ones((16,), jnp.float32)
    plsc.subcore_barrier()

    def body(j, carry):
        base = pl.multiple_of(wid * _EPT + j * _CHUNK, 8)
        pltpu.sync_copy(dst_hbm.at[pl.ds(base, _CHUNK)], idx_v)
        pltpu.sync_copy(ones_v, acc_sh.at[idx_v], add=True)
        return carry

    lax.fori_loop(0, _NCHUNK, body, 0)
    plsc.subcore_barrier()
    @pl.when(s < 5)
    def _():
        pltpu.sync_copy(acc_sh.at[pl.ds(s * 2000, 2000)], zbuf_v)
        pltpu.sync_copy(zbuf_v, out_hbm.at[pl.ds(c * _N + s * 2000, 2000)])


_GK = 4
_NGRP = _NCHUNK // _GK


@functools.partial(
    pl.kernel,
    out_type=jax.ShapeDtypeStruct((_NC, _N, _H), jnp.float32),
    mesh=_mesh,
    scratch_types=[
        pltpu.VMEM((_GK, _CHUNK), jnp.int32),
        pltpu.VMEM((_GK, _CHUNK), jnp.int32),
        pltpu.VMEM((_GK * _CHUNK, _H), jnp.float32),
        pltpu.VMEM_SHARED((_N, _H), jnp.float32),
        pltpu.SemaphoreType.DMA,
        pltpu.SemaphoreType.DMA,
        pltpu.SemaphoreType.DMA,
    ],
)
def _agg_kernel(y_hbm, src_hbm, dst_hbm, out_hbm,
                src_v, dst_v, rows_v, acc_sh, isem, gsem, ssem):
    c = lax.axis_index("c")
    s = lax.axis_index("s")
    wid = c * _NS + s
    r0 = s * _RPT
    def zrow(i, carry):
        for k in range(_H // 16):
            rows_v[i, pl.ds(k * 16, 16)] = jnp.zeros((16,), jnp.float32)
        return carry
    lax.fori_loop(0, _CHUNK, zrow, 0)
    zslot = rows_v.at[pl.ds(0, _CHUNK)]

    @pl.when(s < 15)
    def _():
        for t in range(_RPT // _CHUNK):
            pltpu.sync_copy(zslot, acc_sh.at[pl.ds(r0 + t * _CHUNK, _CHUNK)])
        pltpu.sync_copy(rows_v.at[pl.ds(0, _RPT % _CHUNK)],
                        acc_sh.at[pl.ds(r0 + _RPT - _RPT % _CHUNK,
                                        _RPT % _CHUNK)])
    @pl.when(s == 15)
    def _():
        for t in range(_RPT_LAST // _CHUNK):
            pltpu.sync_copy(zslot, acc_sh.at[pl.ds(r0 + t * _CHUNK, _CHUNK)])
        pltpu.sync_copy(rows_v.at[pl.ds(0, _RPT_LAST % _CHUNK)],
                        acc_sh.at[pl.ds(r0 + _RPT_LAST - _RPT_LAST % _CHUNK,
                                        _RPT_LAST % _CHUNK)])
    plsc.subcore_barrier()

    def slot(k):
        return rows_v.at[pl.ds(k * _CHUNK, _CHUNK)]

    def ebase(j):
        return pl.multiple_of(wid * _EPT + j * _CHUNK, 8)

    def body(g, carry):
        jb = g * _GK
        idescs = []
        for k in range(_GK):
            idescs.append(pltpu.async_copy(
                src_hbm.at[pl.ds(ebase(jb + k), _CHUNK)], src_v.at[k], isem))
            idescs.append(pltpu.async_copy(
                dst_hbm.at[pl.ds(ebase(jb + k), _CHUNK)], dst_v.at[k], isem))
        gdescs = []
        for k in range(_GK):
            idescs[2 * k].wait()
            gdescs.append(
                pltpu.async_copy(y_hbm.at[src_v.at[k]], slot(k), gsem))
        sdescs = []
        for k in range(_GK):
            gdescs[k].wait()
            idescs[2 * k + 1].wait()
            sdescs.append(pltpu.async_copy(slot(k), acc_sh.at[dst_v.at[k]],
                                           ssem, add=True))
        for d in sdescs:
            d.wait()
        return carry

    lax.fori_loop(0, _NGRP, body, 0)
    for j in range(_NGRP * _GK, _NCHUNK):
        pltpu.sync_copy(src_hbm.at[pl.ds(ebase(j), _CHUNK)], src_v.at[0])
        pltpu.sync_copy(dst_hbm.at[pl.ds(ebase(j), _CHUNK)], dst_v.at[0])
        pltpu.async_copy(y_hbm.at[src_v.at[0]], slot(0), gsem).wait()
        pltpu.sync_copy(slot(0), acc_sh.at[dst_v.at[0]], add=True)
    plsc.subcore_barrier()

    def copy_out(nrows):
        stage = rows_v.at[pl.ds(0, _CHUNK)]
        for t in range(nrows // _CHUNK):
            pltpu.sync_copy(acc_sh.at[pl.ds(r0 + t * _CHUNK, _CHUNK)], stage)
            pltpu.sync_copy(stage, out_hbm.at[c, pl.ds(r0 + t * _CHUNK,
                                                       _CHUNK)])
        rem = nrows % _CHUNK
        pltpu.sync_copy(acc_sh.at[pl.ds(r0 + nrows - rem, rem)],
                        rows_v.at[pl.ds(0, rem)])
        pltpu.sync_copy(rows_v.at[pl.ds(0, rem)],
                        out_hbm.at[c, pl.ds(r0 + nrows - rem, rem)])

    @pl.when(s < 15)
    def _():
        copy_out(_RPT)
    @pl.when(s == 15)
    def _():
        copy_out(_RPT_LAST)


_R = 1000
_G = _N // _R


def _lin0_body(x_ref, w_ref, degp_ref, y_ref, dis_ref):
    deg = degp_ref[:, 0:1] + degp_ref[:, 1:2] + 1.0
    dis = lax.rsqrt(deg)
    dis_ref[...] = dis
    y_ref[...] = jnp.dot(x_ref[...], w_ref[...],
                         preferred_element_type=jnp.float32) * dis


def _lin0(x, w, degp_t):
    return pl.pallas_call(
        _lin0_body,
        grid=(_G,),
        in_specs=[
            pl.BlockSpec((_R, _D), lambda i: (i, 0)),
            pl.BlockSpec((_D, _H), lambda i: (0, 0)),
            pl.BlockSpec((_R, 2), lambda i: (i, 0)),
        ],
        out_specs=[
            pl.BlockSpec((_R, _H), lambda i: (i, 0)),
            pl.BlockSpec((_R, 1), lambda i: (i, 0)),
        ],
        out_shape=[
            jax.ShapeDtypeStruct((_N, _H), jnp.float32),
            jax.ShapeDtypeStruct((_N, 1), jnp.float32),
        ],
    )(x, w, degp_t)


def _mid_body(p_ref, y_ref, dis_ref, b_ref, w_ref, y1_ref):
    agg = p_ref[0] + p_ref[1] + y_ref[...]
    dis = dis_ref[...]
    h = jnp.maximum(agg * dis + b_ref[...], 0.0)
    y1_ref[...] = jnp.dot(h, w_ref[...],
                          preferred_element_type=jnp.float32) * dis


def _mid(p, y, dis, b, w):
    return pl.pallas_call(
        _mid_body,
        grid=(_G,),
        in_specs=[
            pl.BlockSpec((2, _R, _H), lambda i: (0, i, 0)),
            pl.BlockSpec((_R, _H), lambda i: (i, 0)),
            pl.BlockSpec((_R, 1), lambda i: (i, 0)),
            pl.BlockSpec((1, _H), lambda i: (0, 0)),
            pl.BlockSpec((_H, _H), lambda i: (0, 0)),
        ],
        out_specs=pl.BlockSpec((_R, _H), lambda i: (i, 0)),
        out_shape=jax.ShapeDtypeStruct((_N, _H), jnp.float32),
    )(p, y, dis, b, w)


def _head_body(q_ref, y_ref, dis_ref, b1_ref, m0_ref, mb0_ref, m1_ref,
               mb1_ref, m2_ref, mb2_ref, o_ref):
    agg = q_ref[0] + q_ref[1] + y_ref[...]
    h = jnp.maximum(agg * dis_ref[...] + b1_ref[...], 0.0)
    h = jnp.maximum(jnp.dot(h, m0_ref[...],
                            preferred_element_type=jnp.float32) + mb0_ref[...], 0.0)
    h = jnp.maximum(jnp.dot(h, m1_ref[...],
                            preferred_element_type=jnp.float32) + mb1_ref[...], 0.0)
    o = jnp.dot(h, m2_ref[...], preferred_element_type=jnp.float32) + mb2_ref[...]
    o_ref[...] = jax.nn.sigmoid(o)


def _head(q, y, dis, b1, m0, mb0, m1, mb1, m2, mb2):
    return pl.pallas_call(
        _head_body,
        grid=(_G,),
        in_specs=[
            pl.BlockSpec((2, _R, _H), lambda i: (0, i, 0)),
            pl.BlockSpec((_R, _H), lambda i: (i, 0)),
            pl.BlockSpec((_R, 1), lambda i: (i, 0)),
            pl.BlockSpec((1, _H), lambda i: (0, 0)),
            pl.BlockSpec((_H, _H), lambda i: (0, 0)),
            pl.BlockSpec((1, _H), lambda i: (0, 0)),
            pl.BlockSpec((_H, _H), lambda i: (0, 0)),
            pl.BlockSpec((1, _H), lambda i: (0, 0)),
            pl.BlockSpec((_H, 1), lambda i: (0, 0)),
            pl.BlockSpec((1, 1), lambda i: (0, 0)),
        ],
        out_specs=pl.BlockSpec((_R, 1), lambda i: (i, 0)),
        out_shape=jax.ShapeDtypeStruct((_N, 1), jnp.float32),
    )(q, y, dis, b1, m0, mb0, m1, mb1, m2, mb2)


def kernel(x, edge_index, W0, b0, W1, b1, M0, mb0, M1, mb1, M2, mb2):
    ei = edge_index.astype(jnp.int32)
    src = ei[0]
    dst = ei[1]
    degp = _deg_kernel(dst)
    degp_t = degp.reshape(_NC, _N).T

    y0, dis = _lin0(x, W0, degp_t)
    p = _agg_kernel(y0, src, dst)
    y1 = _mid(p, y0, dis, b0.reshape(1, _H), W1)
    q = _agg_kernel(y1, src, dst)
    out = _head(q, y1, dis, b1.reshape(1, _H), M0, mb0.reshape(1, _H),
                M1, mb1.reshape(1, _H), M2, mb2.reshape(1, 1))
    return out


# speedup vs baseline: 28.3565x; 1.1281x over previous
import functools

import jax
import jax.numpy as jnp
from jax import lax
from jax.experimental import pallas as pl
from jax.experimental.pallas import tpu as pltpu
from jax.experimental.pallas import tpu_sc as plsc

_N = 10000
_E = 320000
_D = 128
_H = 128

_NC = 2
_NS = 16
_NW = _NC * _NS
_EPT = _E // _NW
_CHUNK = 80
_NCHUNK = _EPT // _CHUNK
_RPT = 632
_RPT_LAST = _N - 15 * _RPT

_mesh = plsc.VectorSubcoreMesh(core_axis_name="c", subcore_axis_name="s")

_GD = 5


@functools.partial(
    pl.kernel,
    out_type=jax.ShapeDtypeStruct((_NC * _N,), jnp.float32),
    mesh=_mesh,
    scratch_types=[
        pltpu.VMEM((_GD, _CHUNK), jnp.int32),
        pltpu.VMEM((_CHUNK,), jnp.float32),
        pltpu.VMEM((2000,), jnp.float32),
        pltpu.VMEM_SHARED((_N,), jnp.float32),
        pltpu.SemaphoreType.DMA,
        pltpu.SemaphoreType.DMA,
    ],
)
def _deg_kernel(dst_hbm, out_hbm, idx_v, ones_v, zbuf_v, acc_sh, isem, ssem):
    c = lax.axis_index("c")
    s = lax.axis_index("s")
    wid = c * _NS + s
    @pl.when(s < 5)
    def _():
        def zfill(i, carry):
            zbuf_v[pl.ds(i * 16, 16)] = jnp.zeros((16,), jnp.float32)
            return carry
        lax.fori_loop(0, 2000 // 16, zfill, 0)
        pltpu.sync_copy(zbuf_v, acc_sh.at[pl.ds(s * 2000, 2000)])
    for k in range(_CHUNK // 16):
        ones_v[pl.ds(k * 16, 16)] = jnp.ones((16,), jnp.float32)
    plsc.subcore_barrier()

    def body(g, carry):
        jb = g * _GD
        idescs = []
        for k in range(_GD):
            base = pl.multiple_of(wid * _EPT + (jb + k) * _CHUNK, 8)
            idescs.append(pltpu.async_copy(
                dst_hbm.at[pl.ds(base, _CHUNK)], idx_v.at[k], isem))
        sdescs = []
        for k in range(_GD):
            idescs[k].wait()
            sdescs.append(pltpu.async_copy(ones_v, acc_sh.at[idx_v.at[k]],
                                           ssem, add=True))
        for d in sdescs:
            d.wait()
        return carry

    lax.fori_loop(0, _NCHUNK // _GD, body, 0)
    plsc.subcore_barrier()
    @pl.when(s < 5)
    def _():
        pltpu.sync_copy(acc_sh.at[pl.ds(s * 2000, 2000)], zbuf_v)
        pltpu.sync_copy(zbuf_v, out_hbm.at[pl.ds(c * _N + s * 2000, 2000)])


_GK = 4
_NGRP = _NCHUNK // _GK


@functools.partial(
    pl.kernel,
    out_type=jax.ShapeDtypeStruct((_NC, _N, _H), jnp.float32),
    mesh=_mesh,
    scratch_types=[
        pltpu.VMEM((_GK, _CHUNK), jnp.int32),
        pltpu.VMEM((_GK, _CHUNK), jnp.int32),
        pltpu.VMEM((_GK * _CHUNK, _H), jnp.float32),
        pltpu.VMEM_SHARED((_N, _H), jnp.float32),
        pltpu.SemaphoreType.DMA,
        pltpu.SemaphoreType.DMA,
        pltpu.SemaphoreType.DMA,
    ],
)
def _agg_kernel(y_hbm, src_hbm, dst_hbm, out_hbm,
                src_v, dst_v, rows_v, acc_sh, isem, gsem, ssem):
    c = lax.axis_index("c")
    s = lax.axis_index("s")
    wid = c * _NS + s
    r0 = s * _RPT
    def zrow(i, carry):
        for k in range(_H // 16):
            rows_v[i, pl.ds(k * 16, 16)] = jnp.zeros((16,), jnp.float32)
        return carry
    lax.fori_loop(0, _CHUNK, zrow, 0)
    zslot = rows_v.at[pl.ds(0, _CHUNK)]

    @pl.when(s < 15)
    def _():
        for t in range(_RPT // _CHUNK):
            pltpu.sync_copy(zslot, acc_sh.at[pl.ds(r0 + t * _CHUNK, _CHUNK)])
        pltpu.sync_copy(rows_v.at[pl.ds(0, _RPT % _CHUNK)],
                        acc_sh.at[pl.ds(r0 + _RPT - _RPT % _CHUNK,
                                        _RPT % _CHUNK)])
    @pl.when(s == 15)
    def _():
        for t in range(_RPT_LAST // _CHUNK):
            pltpu.sync_copy(zslot, acc_sh.at[pl.ds(r0 + t * _CHUNK, _CHUNK)])
        pltpu.sync_copy(rows_v.at[pl.ds(0, _RPT_LAST % _CHUNK)],
                        acc_sh.at[pl.ds(r0 + _RPT_LAST - _RPT_LAST % _CHUNK,
                                        _RPT_LAST % _CHUNK)])
    plsc.subcore_barrier()

    def slot(k):
        return rows_v.at[pl.ds(k * _CHUNK, _CHUNK)]

    def ebase(j):
        return pl.multiple_of(wid * _EPT + j * _CHUNK, 8)

    def body(g, carry):
        jb = g * _GK
        idescs = []
        for k in range(_GK):
            idescs.append(pltpu.async_copy(
                src_hbm.at[pl.ds(ebase(jb + k), _CHUNK)], src_v.at[k], isem))
            idescs.append(pltpu.async_copy(
                dst_hbm.at[pl.ds(ebase(jb + k), _CHUNK)], dst_v.at[k], isem))
        gdescs = []
        for k in range(_GK):
            idescs[2 * k].wait()
            gdescs.append(
                pltpu.async_copy(y_hbm.at[src_v.at[k]], slot(k), gsem))
        sdescs = []
        for k in range(_GK):
            gdescs[k].wait()
            idescs[2 * k + 1].wait()
            sdescs.append(pltpu.async_copy(slot(k), acc_sh.at[dst_v.at[k]],
                                           ssem, add=True))
        for d in sdescs:
            d.wait()
        return carry

    lax.fori_loop(0, _NGRP, body, 0)
    for j in range(_NGRP * _GK, _NCHUNK):
        pltpu.sync_copy(src_hbm.at[pl.ds(ebase(j), _CHUNK)], src_v.at[0])
        pltpu.sync_copy(dst_hbm.at[pl.ds(ebase(j), _CHUNK)], dst_v.at[0])
        pltpu.async_copy(y_hbm.at[src_v.at[0]], slot(0), gsem).wait()
        pltpu.sync_copy(slot(0), acc_sh.at[dst_v.at[0]], add=True)
    plsc.subcore_barrier()

    def copy_out(nrows):
        stage = rows_v.at[pl.ds(0, _CHUNK)]
        for t in range(nrows // _CHUNK):
            pltpu.sync_copy(acc_sh.at[pl.ds(r0 + t * _CHUNK, _CHUNK)], stage)
            pltpu.sync_copy(stage, out_hbm.at[c, pl.ds(r0 + t * _CHUNK,
                                                       _CHUNK)])
        rem = nrows % _CHUNK
        pltpu.sync_copy(acc_sh.at[pl.ds(r0 + nrows - rem, rem)],
                        rows_v.at[pl.ds(0, rem)])
        pltpu.sync_copy(rows_v.at[pl.ds(0, rem)],
                        out_hbm.at[c, pl.ds(r0 + nrows - rem, rem)])

    @pl.when(s < 15)
    def _():
        copy_out(_RPT)
    @pl.when(s == 15)
    def _():
        copy_out(_RPT_LAST)


_R = 1000
_G = _N // _R


def _mm_body(x_ref, w_ref, o_ref):
    o_ref[...] = jnp.dot(x_ref[...], w_ref[...],
                         preferred_element_type=jnp.float32)


def _mm(x, w):
    return pl.pallas_call(
        _mm_body,
        grid=(_G,),
        in_specs=[
            pl.BlockSpec((_R, _D), lambda i: (i, 0)),
            pl.BlockSpec((_D, _H), lambda i: (0, 0)),
        ],
        out_specs=pl.BlockSpec((_R, _H), lambda i: (i, 0)),
        out_shape=jax.ShapeDtypeStruct((_N, _H), jnp.float32),
    )(x, w)


def _scale_body(xw_ref, degp_ref, y_ref, dis_ref):
    deg = degp_ref[:, 0:1] + degp_ref[:, 1:2] + 1.0
    dis = lax.rsqrt(deg)
    dis_ref[...] = dis
    y_ref[...] = xw_ref[...] * dis


def _scale(xw, degp_t):
    return pl.pallas_call(
        _scale_body,
        grid=(_G,),
        in_specs=[
            pl.BlockSpec((_R, _H), lambda i: (i, 0)),
            pl.BlockSpec((_R, 2), lambda i: (i, 0)),
        ],
        out_specs=[
            pl.BlockSpec((_R, _H), lambda i: (i, 0)),
            pl.BlockSpec((_R, 1), lambda i: (i, 0)),
        ],
        out_shape=[
            jax.ShapeDtypeStruct((_N, _H), jnp.float32),
            jax.ShapeDtypeStruct((_N, 1), jnp.float32),
        ],
    )(xw, degp_t)


def _mid_body(p_ref, y_ref, dis_ref, b_ref, w_ref, y1_ref):
    agg = p_ref[0] + p_ref[1] + y_ref[...]
    dis = dis_ref[...]
    h = jnp.maximum(agg * dis + b_ref[...], 0.0)
    y1_ref[...] = jnp.dot(h, w_ref[...],
                          preferred_element_type=jnp.float32) * dis


def _mid(p, y, dis, b, w):
    return pl.pallas_call(
        _mid_body,
        grid=(_G,),
        in_specs=[
            pl.BlockSpec((2, _R, _H), lambda i: (0, i, 0)),
            pl.BlockSpec((_R, _H), lambda i: (i, 0)),
            pl.BlockSpec((_R, 1), lambda i: (i, 0)),
            pl.BlockSpec((1, _H), lambda i: (0, 0)),
            pl.BlockSpec((_H, _H), lambda i: (0, 0)),
        ],
        out_specs=pl.BlockSpec((_R, _H), lambda i: (i, 0)),
        out_shape=jax.ShapeDtypeStruct((_N, _H), jnp.float32),
    )(p, y, dis, b, w)


def _head_body(q_ref, y_ref, dis_ref, b1_ref, m0_ref, mb0_ref, m1_ref,
               mb1_ref, m2_ref, mb2_ref, o_ref):
    agg = q_ref[0] + q_ref[1] + y_ref[...]
    h = jnp.maximum(agg * dis_ref[...] + b1_ref[...], 0.0)
    h = jnp.maximum(jnp.dot(h, m0_ref[...],
                            preferred_element_type=jnp.float32) + mb0_ref[...], 0.0)
    h = jnp.maximum(jnp.dot(h, m1_ref[...],
                            preferred_element_type=jnp.float32) + mb1_ref[...], 0.0)
    o = jnp.dot(h, m2_ref[...], preferred_element_type=jnp.float32) + mb2_ref[...]
    o_ref[...] = jax.nn.sigmoid(o)


def _head(q, y, dis, b1, m0, mb0, m1, mb1, m2, mb2):
    return pl.pallas_call(
        _head_body,
        grid=(_G,),
        in_specs=[
            pl.BlockSpec((2, _R, _H), lambda i: (0, i, 0)),
            pl.BlockSpec((_R, _H), lambda i: (i, 0)),
            pl.BlockSpec((_R, 1), lambda i: (i, 0)),
            pl.BlockSpec((1, _H), lambda i: (0, 0)),
            pl.BlockSpec((_H, _H), lambda i: (0, 0)),
            pl.BlockSpec((1, _H), lambda i: (0, 0)),
            pl.BlockSpec((_H, _H), lambda i: (0, 0)),
            pl.BlockSpec((1, _H), lambda i: (0, 0)),
            pl.BlockSpec((_H, 1), lambda i: (0, 0)),
            pl.BlockSpec((1, 1), lambda i: (0, 0)),
        ],
        out_specs=pl.BlockSpec((_R, 1), lambda i: (i, 0)),
        out_shape=jax.ShapeDtypeStruct((_N, 1), jnp.float32),
    )(q, y, dis, b1, m0, mb0, m1, mb1, m2, mb2)


def kernel(x, edge_index, W0, b0, W1, b1, M0, mb0, M1, mb1, M2, mb2):
    ei = edge_index.astype(jnp.int32)
    src = ei[0]
    dst = ei[1]
    xw0 = _mm(x, W0)
    degp = _deg_kernel(dst)
    degp_t = degp.reshape(_NC, _N).T

    y0, dis = _scale(xw0, degp_t)
    p = _agg_kernel(y0, src, dst)
    y1 = _mid(p, y0, dis, b0.reshape(1, _H), W1)
    q = _agg_kernel(y1, src, dst)
    out = _head(q, y1, dis, b1.reshape(1, _H), M0, mb0.reshape(1, _H),
                M1, mb1.reshape(1, _H), M2, mb2.reshape(1, 1))
    return out


# speedup vs baseline: 30.2936x; 1.0683x over previous
import functools

import jax
import jax.numpy as jnp
from jax import lax
from jax.experimental import pallas as pl
from jax.experimental.pallas import tpu as pltpu
from jax.experimental.pallas import tpu_sc as plsc

_N = 10000
_E = 320000
_D = 128
_H = 128

_NC = 2
_NS = 16
_NW = _NC * _NS
_EPT = _E // _NW
_CHUNK = 80
_NCHUNK = _EPT // _CHUNK
_RPT = 632
_RPT_LAST = _N - 15 * _RPT

_mesh = plsc.VectorSubcoreMesh(core_axis_name="c", subcore_axis_name="s")

_GD = 5


@functools.partial(
    pl.kernel,
    out_type=jax.ShapeDtypeStruct((_NC * _N,), jnp.float32),
    mesh=_mesh,
    scratch_types=[
        pltpu.VMEM((_GD, _CHUNK), jnp.int32),
        pltpu.VMEM((_CHUNK,), jnp.float32),
        pltpu.VMEM((2000,), jnp.float32),
        pltpu.VMEM_SHARED((_N,), jnp.float32),
        pltpu.SemaphoreType.DMA,
        pltpu.SemaphoreType.DMA,
    ],
)
def _deg_kernel(dst_hbm, out_hbm, idx_v, ones_v, zbuf_v, acc_sh, isem, ssem):
    c = lax.axis_index("c")
    s = lax.axis_index("s")
    wid = c * _NS + s
    @pl.when(s < 5)
    def _():
        def zfill(i, carry):
            zbuf_v[pl.ds(i * 16, 16)] = jnp.zeros((16,), jnp.float32)
            return carry
        lax.fori_loop(0, 2000 // 16, zfill, 0)
        pltpu.sync_copy(zbuf_v, acc_sh.at[pl.ds(s * 2000, 2000)])
    for k in range(_CHUNK // 16):
        ones_v[pl.ds(k * 16, 16)] = jnp.ones((16,), jnp.float32)
    plsc.subcore_barrier()

    def body(g, carry):
        jb = g * _GD
        idescs = []
        for k in range(_GD):
            base = pl.multiple_of(wid * _EPT + (jb + k) * _CHUNK, 8)
            idescs.append(pltpu.async_copy(
                dst_hbm.at[pl.ds(base, _CHUNK)], idx_v.at[k], isem))
        sdescs = []
        for k in range(_GD):
            idescs[k].wait()
            sdescs.append(pltpu.async_copy(ones_v, acc_sh.at[idx_v.at[k]],
                                           ssem, add=True))
        for d in sdescs:
            d.wait()
        return carry

    lax.fori_loop(0, _NCHUNK // _GD, body, 0)
    plsc.subcore_barrier()
    @pl.when(s < 5)
    def _():
        pltpu.sync_copy(acc_sh.at[pl.ds(s * 2000, 2000)], zbuf_v)
        pltpu.sync_copy(zbuf_v, out_hbm.at[pl.ds(c * _N + s * 2000, 2000)])


_NSLOT = 4
_NISLOT = 8
_NGRP = _NCHUNK // 2


@functools.partial(
    pl.kernel,
    out_type=jax.ShapeDtypeStruct((_NC, _N, _H), jnp.float32),
    mesh=_mesh,
    scratch_types=[
        pltpu.VMEM((_NISLOT, _CHUNK), jnp.int32),
        pltpu.VMEM((_NISLOT, _CHUNK), jnp.int32),
        pltpu.VMEM((_NSLOT * _CHUNK, _H), jnp.float32),
        pltpu.VMEM_SHARED((_N, _H), jnp.float32),
        pltpu.SemaphoreType.DMA,
        pltpu.SemaphoreType.DMA,
        pltpu.SemaphoreType.DMA,
    ],
)
def _agg_kernel(y_hbm, src_hbm, dst_hbm, out_hbm,
                src_v, dst_v, rows_v, acc_sh, isem, gsem, ssem):
    c = lax.axis_index("c")
    s = lax.axis_index("s")
    wid = c * _NS + s
    r0 = s * _RPT
    def zrow(i, carry):
        for k in range(_H // 16):
            rows_v[i, pl.ds(k * 16, 16)] = jnp.zeros((16,), jnp.float32)
        return carry
    lax.fori_loop(0, _CHUNK, zrow, 0)
    zslot = rows_v.at[pl.ds(0, _CHUNK)]

    @pl.when(s < 15)
    def _():
        for t in range(_RPT // _CHUNK):
            pltpu.sync_copy(zslot, acc_sh.at[pl.ds(r0 + t * _CHUNK, _CHUNK)])
        pltpu.sync_copy(rows_v.at[pl.ds(0, _RPT % _CHUNK)],
                        acc_sh.at[pl.ds(r0 + _RPT - _RPT % _CHUNK,
                                        _RPT % _CHUNK)])
    @pl.when(s == 15)
    def _():
        for t in range(_RPT_LAST // _CHUNK):
            pltpu.sync_copy(zslot, acc_sh.at[pl.ds(r0 + t * _CHUNK, _CHUNK)])
        pltpu.sync_copy(rows_v.at[pl.ds(0, _RPT_LAST % _CHUNK)],
                        acc_sh.at[pl.ds(r0 + _RPT_LAST - _RPT_LAST % _CHUNK,
                                        _RPT_LAST % _CHUNK)])
    plsc.subcore_barrier()

    def slot(j):
        return rows_v.at[pl.ds(lax.rem(j, _NSLOT) * _CHUNK, _CHUNK)]

    def ebase(j):
        return pl.multiple_of(wid * _EPT + j * _CHUNK, 8)

    def fire_idx(j):
        jc = lax.min(j, _NCHUNK - 2)
        sl = lax.rem(jc, _NISLOT)
        pltpu.async_copy(src_hbm.at[pl.ds(ebase(jc), _CHUNK)],
                         src_v.at[sl], isem)
        pltpu.async_copy(dst_hbm.at[pl.ds(ebase(jc), _CHUNK)],
                         dst_v.at[sl], isem)

    def wait_idx():
        for _ in range(2):
            pltpu.make_async_copy(src_hbm.at[pl.ds(0, _CHUNK)],
                                  src_v.at[0], isem).wait()

    def fire_gather(j):
        return pltpu.async_copy(
            y_hbm.at[src_v.at[lax.rem(j, _NISLOT)]], slot(j), gsem)

    def wait_gather():
        pltpu.make_async_copy(y_hbm.at[pl.ds(0, _CHUNK)],
                              rows_v.at[pl.ds(0, _CHUNK)], gsem).wait()

    def fire_scatter(j):
        pltpu.async_copy(slot(j), acc_sh.at[dst_v.at[lax.rem(j, _NISLOT)]],
                         ssem, add=True)

    def wait_scatter():
        pltpu.make_async_copy(y_hbm.at[pl.ds(0, _CHUNK)],
                              rows_v.at[pl.ds(0, _CHUNK)], ssem).wait()

    for j in range(2 * _NGRP, _NCHUNK):
        pltpu.sync_copy(src_hbm.at[pl.ds(ebase(j), _CHUNK)], src_v.at[0])
        pltpu.sync_copy(dst_hbm.at[pl.ds(ebase(j), _CHUNK)], dst_v.at[0])
        pltpu.async_copy(y_hbm.at[src_v.at[0]], slot(0), gsem).wait()
        pltpu.sync_copy(slot(0), acc_sh.at[dst_v.at[0]], add=True)

    for j in range(6):
        fire_idx(j)
    wait_idx(); wait_idx()
    g0 = fire_gather(0)
    g1 = fire_gather(1)
    g0.wait()
    g1.wait()
    fire_scatter(0)
    fire_scatter(1)
    wait_idx(); wait_idx()
    fire_gather(2)
    fire_gather(3)

    def body(g, carry):
        jb = 2 * g
        wait_scatter(); wait_scatter()
        wait_gather(); wait_gather()
        fire_scatter(jb)
        fire_scatter(jb + 1)
        wait_idx(); wait_idx()
        fire_gather(jb + 2)
        fire_gather(jb + 3)
        fire_idx(jb + 4)
        fire_idx(jb + 5)
        return carry

    lax.fori_loop(1, _NGRP - 1, body, 0)
    wait_scatter(); wait_scatter()
    wait_gather(); wait_gather()
    fire_scatter(2 * _NGRP - 2)
    fire_scatter(2 * _NGRP - 1)
    wait_idx(); wait_idx()
    wait_scatter(); wait_scatter()
    plsc.subcore_barrier()

    def copy_out(nrows):
        stage = rows_v.at[pl.ds(0, _CHUNK)]
        for t in range(nrows // _CHUNK):
            pltpu.sync_copy(acc_sh.at[pl.ds(r0 + t * _CHUNK, _CHUNK)], stage)
            pltpu.sync_copy(stage, out_hbm.at[c, pl.ds(r0 + t * _CHUNK,
                                                       _CHUNK)])
        rem = nrows % _CHUNK
        pltpu.sync_copy(acc_sh.at[pl.ds(r0 + nrows - rem, rem)],
                        rows_v.at[pl.ds(0, rem)])
        pltpu.sync_copy(rows_v.at[pl.ds(0, rem)],
                        out_hbm.at[c, pl.ds(r0 + nrows - rem, rem)])

    @pl.when(s < 15)
    def _():
        copy_out(_RPT)
    @pl.when(s == 15)
    def _():
        copy_out(_RPT_LAST)


_R = 1000
_G = _N // _R


def _mm_body(x_ref, w_ref, o_ref):
    o_ref[...] = jnp.dot(x_ref[...], w_ref[...],
                         preferred_element_type=jnp.float32)


def _mm(x, w):
    return pl.pallas_call(
        _mm_body,
        grid=(_G,),
        in_specs=[
            pl.BlockSpec((_R, _D), lambda i: (i, 0)),
            pl.BlockSpec((_D, _H), lambda i: (0, 0)),
        ],
        out_specs=pl.BlockSpec((_R, _H), lambda i: (i, 0)),
        out_shape=jax.ShapeDtypeStruct((_N, _H), jnp.float32),
    )(x, w)


def _scale_body(xw_ref, degp_ref, y_ref, dis_ref):
    deg = degp_ref[:, 0:1] + degp_ref[:, 1:2] + 1.0
    dis = lax.rsqrt(deg)
    dis_ref[...] = dis
    y_ref[...] = xw_ref[...] * dis


def _scale(xw, degp_t):
    return pl.pallas_call(
        _scale_body,
        grid=(_G,),
        in_specs=[
            pl.BlockSpec((_R, _H), lambda i: (i, 0)),
            pl.BlockSpec((_R, 2), lambda i: (i, 0)),
        ],
        out_specs=[
            pl.BlockSpec((_R, _H), lambda i: (i, 0)),
            pl.BlockSpec((_R, 1), lambda i: (i, 0)),
        ],
        out_shape=[
            jax.ShapeDtypeStruct((_N, _H), jnp.float32),
            jax.ShapeDtypeStruct((_N, 1), jnp.float32),
        ],
    )(xw, degp_t)


def _mid_body(p_ref, y_ref, dis_ref, b_ref, w_ref, y1_ref):
    agg = p_ref[0] + p_ref[1] + y_ref[...]
    dis = dis_ref[...]
    h = jnp.maximum(agg * dis + b_ref[...], 0.0)
    y1_ref[...] = jnp.dot(h, w_ref[...],
                          preferred_element_type=jnp.float32) * dis


def _mid(p, y, dis, b, w):
    return pl.pallas_call(
        _mid_body,
        grid=(_G,),
        in_specs=[
            pl.BlockSpec((2, _R, _H), lambda i: (0, i, 0)),
            pl.BlockSpec((_R, _H), lambda i: (i, 0)),
            pl.BlockSpec((_R, 1), lambda i: (i, 0)),
            pl.BlockSpec((1, _H), lambda i: (0, 0)),
            pl.BlockSpec((_H, _H), lambda i: (0, 0)),
        ],
        out_specs=pl.BlockSpec((_R, _H), lambda i: (i, 0)),
        out_shape=jax.ShapeDtypeStruct((_N, _H), jnp.float32),
    )(p, y, dis, b, w)


def _head_body(q_ref, y_ref, dis_ref, b1_ref, m0_ref, mb0_ref, m1_ref,
               mb1_ref, m2_ref, mb2_ref, o_ref):
    agg = q_ref[0] + q_ref[1] + y_ref[...]
    h = jnp.maximum(agg * dis_ref[...] + b1_ref[...], 0.0)
    h = jnp.maximum(jnp.dot(h, m0_ref[...],
                            preferred_element_type=jnp.float32) + mb0_ref[...], 0.0)
    h = jnp.maximum(jnp.dot(h, m1_ref[...],
                            preferred_element_type=jnp.float32) + mb1_ref[...], 0.0)
    o = jnp.dot(h, m2_ref[...], preferred_element_type=jnp.float32) + mb2_ref[...]
    o_ref[...] = jax.nn.sigmoid(o)


def _head(q, y, dis, b1, m0, mb0, m1, mb1, m2, mb2):
    return pl.pallas_call(
        _head_body,
        grid=(_G,),
        in_specs=[
            pl.BlockSpec((2, _R, _H), lambda i: (0, i, 0)),
            pl.BlockSpec((_R, _H), lambda i: (i, 0)),
            pl.BlockSpec((_R, 1), lambda i: (i, 0)),
            pl.BlockSpec((1, _H), lambda i: (0, 0)),
            pl.BlockSpec((_H, _H), lambda i: (0, 0)),
            pl.BlockSpec((1, _H), lambda i: (0, 0)),
            pl.BlockSpec((_H, _H), lambda i: (0, 0)),
            pl.BlockSpec((1, _H), lambda i: (0, 0)),
            pl.BlockSpec((_H, 1), lambda i: (0, 0)),
            pl.BlockSpec((1, 1), lambda i: (0, 0)),
        ],
        out_specs=pl.BlockSpec((_R, 1), lambda i: (i, 0)),
        out_shape=jax.ShapeDtypeStruct((_N, 1), jnp.float32),
    )(q, y, dis, b1, m0, mb0, m1, mb1, m2, mb2)


def kernel(x, edge_index, W0, b0, W1, b1, M0, mb0, M1, mb1, M2, mb2):
    ei = edge_index.astype(jnp.int32)
    src = ei[0]
    dst = ei[1]
    xw0 = _mm(x, W0)
    degp = _deg_kernel(dst)
    degp_t = degp.reshape(_NC, _N).T

    y0, dis = _scale(xw0, degp_t)
    p = _agg_kernel(y0, src, dst)
    y1 = _mid(p, y0, dis, b0.reshape(1, _H), W1)
    q = _agg_kernel(y1, src, dst)
    out = _head(q, y1, dis, b1.reshape(1, _H), M0, mb0.reshape(1, _H),
                M1, mb1.reshape(1, _H), M2, mb2.reshape(1, 1))
    return out


# speedup vs baseline: 30.9375x; 1.0213x over previous
import functools

import jax
import jax.numpy as jnp
from jax import lax
from jax.experimental import pallas as pl
from jax.experimental.pallas import tpu as pltpu
from jax.experimental.pallas import tpu_sc as plsc

_N = 10000
_E = 320000
_D = 128
_H = 128

_NC = 2
_NS = 16
_NW = _NC * _NS
_EPT = _E // _NW
_CHUNK = 80
_NCHUNK = _EPT // _CHUNK
_RPT = 632
_RPT_LAST = _N - 15 * _RPT

_mesh = plsc.VectorSubcoreMesh(core_axis_name="c", subcore_axis_name="s")

_GD = 5


@functools.partial(
    pl.kernel,
    out_type=jax.ShapeDtypeStruct((_NC * _N,), jnp.float32),
    mesh=_mesh,
    scratch_types=[
        pltpu.VMEM((_GD, _CHUNK), jnp.int32),
        pltpu.VMEM((_CHUNK,), jnp.float32),
        pltpu.VMEM((2000,), jnp.float32),
        pltpu.VMEM_SHARED((_N,), jnp.float32),
        pltpu.SemaphoreType.DMA,
        pltpu.SemaphoreType.DMA,
    ],
)
def _deg_kernel(dst_hbm, out_hbm, idx_v, ones_v, zbuf_v, acc_sh, isem, ssem):
    c = lax.axis_index("c")
    s = lax.axis_index("s")
    wid = c * _NS + s
    @pl.when(s < 5)
    def _():
        def zfill(i, carry):
            zbuf_v[pl.ds(i * 16, 16)] = jnp.zeros((16,), jnp.float32)
            return carry
        lax.fori_loop(0, 2000 // 16, zfill, 0)
        pltpu.sync_copy(zbuf_v, acc_sh.at[pl.ds(s * 2000, 2000)])
    for k in range(_CHUNK // 16):
        ones_v[pl.ds(k * 16, 16)] = jnp.ones((16,), jnp.float32)
    plsc.subcore_barrier()

    def body(g, carry):
        jb = g * _GD
        idescs = []
        for k in range(_GD):
            base = pl.multiple_of(wid * _EPT + (jb + k) * _CHUNK, 8)
            idescs.append(pltpu.async_copy(
                dst_hbm.at[pl.ds(base, _CHUNK)], idx_v.at[k], isem))
        sdescs = []
        for k in range(_GD):
            idescs[k].wait()
            sdescs.append(pltpu.async_copy(ones_v, acc_sh.at[idx_v.at[k]],
                                           ssem, add=True))
        for d in sdescs:
            d.wait()
        return carry

    lax.fori_loop(0, _NCHUNK // _GD, body, 0)
    plsc.subcore_barrier()
    @pl.when(s < 5)
    def _():
        pltpu.sync_copy(acc_sh.at[pl.ds(s * 2000, 2000)], zbuf_v)
        pltpu.sync_copy(zbuf_v, out_hbm.at[pl.ds(c * _N + s * 2000, 2000)])


_NSLOT = 4
_NISLOT = 8
_NGRP = _NCHUNK // 2


@functools.partial(
    pl.kernel,
    out_type=jax.ShapeDtypeStruct((_NC, _N, _H), jnp.float32),
    mesh=_mesh,
    scratch_types=[
        pltpu.VMEM((_NISLOT, _CHUNK), jnp.int32),
        pltpu.VMEM((_NISLOT, _CHUNK), jnp.int32),
        pltpu.VMEM((_NSLOT * _CHUNK, _H), jnp.float32),
        pltpu.VMEM_SHARED((_N, _H), jnp.float32),
        pltpu.SemaphoreType.DMA,
        pltpu.SemaphoreType.DMA,
        pltpu.SemaphoreType.DMA,
    ],
)
def _agg_kernel(y_hbm, src_hbm, dst_hbm, out_hbm,
                src_v, dst_v, rows_v, acc_sh, isem, gsem, ssem):
    c = lax.axis_index("c")
    s = lax.axis_index("s")
    wid = c * _NS + s
    r0 = s * _RPT
    def zrow(i, carry):
        for k in range(_H // 16):
            rows_v[i, pl.ds(k * 16, 16)] = jnp.zeros((16,), jnp.float32)
        return carry
    lax.fori_loop(0, _CHUNK, zrow, 0)
    zslot = rows_v.at[pl.ds(0, _CHUNK)]

    def zero_acc(nrows):
        zdescs = []
        for t in range(nrows // _CHUNK):
            zdescs.append(pltpu.async_copy(
                zslot, acc_sh.at[pl.ds(r0 + t * _CHUNK, _CHUNK)], gsem))
        rem = nrows % _CHUNK
        zdescs.append(pltpu.async_copy(
            rows_v.at[pl.ds(0, rem)],
            acc_sh.at[pl.ds(r0 + nrows - rem, rem)], gsem))
        for d in zdescs:
            d.wait()

    @pl.when(s < 15)
    def _():
        zero_acc(_RPT)
    @pl.when(s == 15)
    def _():
        zero_acc(_RPT_LAST)
    plsc.subcore_barrier()

    def slot(j):
        return rows_v.at[pl.ds(lax.rem(j, _NSLOT) * _CHUNK, _CHUNK)]

    def ebase(j):
        return pl.multiple_of(wid * _EPT + j * _CHUNK, 8)

    def fire_idx(j):
        jc = lax.min(j, _NCHUNK - 2)
        sl = lax.rem(jc, _NISLOT)
        pltpu.async_copy(src_hbm.at[pl.ds(ebase(jc), _CHUNK)],
                         src_v.at[sl], isem)
        pltpu.async_copy(dst_hbm.at[pl.ds(ebase(jc), _CHUNK)],
                         dst_v.at[sl], isem)

    def wait_idx():
        for _ in range(2):
            pltpu.make_async_copy(src_hbm.at[pl.ds(0, _CHUNK)],
                                  src_v.at[0], isem).wait()

    def fire_gather(j):
        return pltpu.async_copy(
            y_hbm.at[src_v.at[lax.rem(j, _NISLOT)]], slot(j), gsem)

    def wait_gather():
        pltpu.make_async_copy(y_hbm.at[pl.ds(0, _CHUNK)],
                              rows_v.at[pl.ds(0, _CHUNK)], gsem).wait()

    def fire_scatter(j):
        pltpu.async_copy(slot(j), acc_sh.at[dst_v.at[lax.rem(j, _NISLOT)]],
                         ssem, add=True)

    def wait_scatter():
        pltpu.make_async_copy(y_hbm.at[pl.ds(0, _CHUNK)],
                              rows_v.at[pl.ds(0, _CHUNK)], ssem).wait()

    for j in range(2 * _NGRP, _NCHUNK):
        pltpu.sync_copy(src_hbm.at[pl.ds(ebase(j), _CHUNK)], src_v.at[0])
        pltpu.sync_copy(dst_hbm.at[pl.ds(ebase(j), _CHUNK)], dst_v.at[0])
        pltpu.async_copy(y_hbm.at[src_v.at[0]], slot(0), gsem).wait()
        pltpu.sync_copy(slot(0), acc_sh.at[dst_v.at[0]], add=True)

    for j in range(6):
        fire_idx(j)
    wait_idx(); wait_idx()
    g0 = fire_gather(0)
    g1 = fire_gather(1)
    g0.wait()
    g1.wait()
    fire_scatter(0)
    fire_scatter(1)
    wait_idx(); wait_idx()
    fire_gather(2)
    fire_gather(3)

    def body(g, carry):
        jb = 2 * g
        wait_scatter(); wait_scatter()
        wait_gather(); wait_gather()
        fire_scatter(jb)
        fire_scatter(jb + 1)
        wait_idx(); wait_idx()
        fire_gather(jb + 2)
        fire_gather(jb + 3)
        fire_idx(jb + 4)
        fire_idx(jb + 5)
        return carry

    lax.fori_loop(1, _NGRP - 1, body, 0)
    wait_scatter(); wait_scatter()
    wait_gather(); wait_gather()
    fire_scatter(2 * _NGRP - 2)
    fire_scatter(2 * _NGRP - 1)
    wait_idx(); wait_idx()
    wait_scatter(); wait_scatter()
    plsc.subcore_barrier()

    def copy_out(nrows):
        hops = [(t * _CHUNK, _CHUNK) for t in range(nrows // _CHUNK)]
        hops.append((nrows - nrows % _CHUNK, nrows % _CHUNK))
        odescs = []
        for t, (off, n) in enumerate(hops):
            stage = rows_v.at[pl.ds((t % 2) * _CHUNK, n)]
            if t >= 2:
                odescs[t - 2].wait()
            pltpu.sync_copy(acc_sh.at[pl.ds(r0 + off, n)], stage)
            odescs.append(pltpu.async_copy(
                stage, out_hbm.at[c, pl.ds(r0 + off, n)], gsem))
        for d in odescs[-2:]:
            d.wait()

    @pl.when(s < 15)
    def _():
        copy_out(_RPT)
    @pl.when(s == 15)
    def _():
        copy_out(_RPT_LAST)


_R = 1000
_G = _N // _R


def _mm_body(x_ref, w_ref, o_ref):
    o_ref[...] = jnp.dot(x_ref[...], w_ref[...],
                         preferred_element_type=jnp.float32)


def _mm(x, w):
    return pl.pallas_call(
        _mm_body,
        grid=(_G,),
        in_specs=[
            pl.BlockSpec((_R, _D), lambda i: (i, 0)),
            pl.BlockSpec((_D, _H), lambda i: (0, 0)),
        ],
        out_specs=pl.BlockSpec((_R, _H), lambda i: (i, 0)),
        out_shape=jax.ShapeDtypeStruct((_N, _H), jnp.float32),
    )(x, w)


def _scale_body(xw_ref, degp_ref, y_ref, dis_ref):
    deg = degp_ref[:, 0:1] + degp_ref[:, 1:2] + 1.0
    dis = lax.rsqrt(deg)
    dis_ref[...] = dis
    y_ref[...] = xw_ref[...] * dis


def _scale(xw, degp_t):
    return pl.pallas_call(
        _scale_body,
        grid=(_G,),
        in_specs=[
            pl.BlockSpec((_R, _H), lambda i: (i, 0)),
            pl.BlockSpec((_R, 2), lambda i: (i, 0)),
        ],
        out_specs=[
            pl.BlockSpec((_R, _H), lambda i: (i, 0)),
            pl.BlockSpec((_R, 1), lambda i: (i, 0)),
        ],
        out_shape=[
            jax.ShapeDtypeStruct((_N, _H), jnp.float32),
            jax.ShapeDtypeStruct((_N, 1), jnp.float32),
        ],
    )(xw, degp_t)


def _mid_body(p_ref, y_ref, dis_ref, b_ref, w_ref, y1_ref):
    agg = p_ref[0] + p_ref[1] + y_ref[...]
    dis = dis_ref[...]
    h = jnp.maximum(agg * dis + b_ref[...], 0.0)
    y1_ref[...] = jnp.dot(h, w_ref[...],
                          preferred_element_type=jnp.float32) * dis


def _mid(p, y, dis, b, w):
    return pl.pallas_call(
        _mid_body,
        grid=(_G,),
        in_specs=[
            pl.BlockSpec((2, _R, _H), lambda i: (0, i, 0)),
            pl.BlockSpec((_R, _H), lambda i: (i, 0)),
            pl.BlockSpec((_R, 1), lambda i: (i, 0)),
            pl.BlockSpec((1, _H), lambda i: (0, 0)),
            pl.BlockSpec((_H, _H), lambda i: (0, 0)),
        ],
        out_specs=pl.BlockSpec((_R, _H), lambda i: (i, 0)),
        out_shape=jax.ShapeDtypeStruct((_N, _H), jnp.float32),
    )(p, y, dis, b, w)


def _head_body(q_ref, y_ref, dis_ref, b1_ref, m0_ref, mb0_ref, m1_ref,
               mb1_ref, m2_ref, mb2_ref, o_ref):
    agg = q_ref[0] + q_ref[1] + y_ref[...]
    h = jnp.maximum(agg * dis_ref[...] + b1_ref[...], 0.0)
    h = jnp.maximum(jnp.dot(h, m0_ref[...],
                            preferred_element_type=jnp.float32) + mb0_ref[...], 0.0)
    h = jnp.maximum(jnp.dot(h, m1_ref[...],
                            preferred_element_type=jnp.float32) + mb1_ref[...], 0.0)
    o = jnp.dot(h, m2_ref[...], preferred_element_type=jnp.float32) + mb2_ref[...]
    o_ref[...] = jax.nn.sigmoid(o)


def _head(q, y, dis, b1, m0, mb0, m1, mb1, m2, mb2):
    return pl.pallas_call(
        _head_body,
        grid=(_G,),
        in_specs=[
            pl.BlockSpec((2, _R, _H), lambda i: (0, i, 0)),
            pl.BlockSpec((_R, _H), lambda i: (i, 0)),
            pl.BlockSpec((_R, 1), lambda i: (i, 0)),
            pl.BlockSpec((1, _H), lambda i: (0, 0)),
            pl.BlockSpec((_H, _H), lambda i: (0, 0)),
            pl.BlockSpec((1, _H), lambda i: (0, 0)),
            pl.BlockSpec((_H, _H), lambda i: (0, 0)),
            pl.BlockSpec((1, _H), lambda i: (0, 0)),
            pl.BlockSpec((_H, 1), lambda i: (0, 0)),
            pl.BlockSpec((1, 1), lambda i: (0, 0)),
        ],
        out_specs=pl.BlockSpec((_R, 1), lambda i: (i, 0)),
        out_shape=jax.ShapeDtypeStruct((_N, 1), jnp.float32),
    )(q, y, dis, b1, m0, mb0, m1, mb1, m2, mb2)


def kernel(x, edge_index, W0, b0, W1, b1, M0, mb0, M1, mb1, M2, mb2):
    ei = edge_index.astype(jnp.int32)
    src = ei[0]
    dst = ei[1]
    xw0 = _mm(x, W0)
    degp = _deg_kernel(dst)
    degp_t = degp.reshape(_NC, _N).T

    y0, dis = _scale(xw0, degp_t)
    p = _agg_kernel(y0, src, dst)
    y1 = _mid(p, y0, dis, b0.reshape(1, _H), W1)
    q = _agg_kernel(y1, src, dst)
    out = _head(q, y1, dis, b1.reshape(1, _H), M0, mb0.reshape(1, _H),
                M1, mb1.reshape(1, _H), M2, mb2.reshape(1, 1))
    return out
